# Initial kernel scaffold; baseline (speedup 1.0000x reference)
import jax
import jax.numpy as jnp
from jax.experimental import pallas as pl


def kernel(x, edge_index, edge_attr, params):
    raise NotImplementedError("write your pallas kernel here")


# speedup vs baseline: 1.7212x; 1.7212x over previous
import functools

import jax
import jax.numpy as jnp
from jax import lax
from jax.experimental import pallas as pl
from jax.experimental.pallas import tpu as pltpu
from jax.experimental.pallas import tpu_sc as plsc

N_NODES = 100000
HID = 64
NE = 1600000

CHUNK = 512
NCHUNK = 98
PER_TILE = CHUNK * NCHUNK
NEP = PER_TILE * 32
JUNK = N_NODES
AGG_ROWS = 100096
ZCH = 184
ZIT = 34

BN = 2000
BE = 2048


def _dot(a, b):
    return jnp.dot(a, b, preferred_element_type=jnp.float32)


def _lin_body(x_ref, w_ref, b_ref, o_ref):
    o_ref[...] = _dot(x_ref[...], w_ref[...]) + b_ref[...]


def _mlp_body(x_ref, w1_ref, b1_ref, w2_ref, b2_ref, o_ref):
    hdn = jnp.maximum(_dot(x_ref[...], w1_ref[...]) + b1_ref[...], 0.0)
    o_ref[...] = _dot(hdn, w2_ref[...]) + b2_ref[...]


def _full(shape):
    return pl.BlockSpec(shape, lambda *i: (0,) * len(shape))


def _lin(x, w, b):
    n, din = x.shape
    dout = w.shape[1]
    grid = n // BN
    return pl.pallas_call(
        _lin_body,
        grid=(grid,),
        in_specs=[pl.BlockSpec((BN, din), lambda i: (i, 0)),
                  _full(w.shape), _full(b.shape)],
        out_specs=pl.BlockSpec((BN, dout), lambda i: (i, 0)),
        out_shape=jax.ShapeDtypeStruct((n, dout), jnp.float32),
    )(x, w, b)


def _mlp_nodes(x, w1, b1, w2, b2):
    n, din = x.shape
    dout = w2.shape[1]
    grid = n // BN
    return pl.pallas_call(
        _mlp_body,
        grid=(grid,),
        in_specs=[pl.BlockSpec((BN, din), lambda i: (i, 0)),
                  _full(w1.shape), _full(b1.shape),
                  _full(w2.shape), _full(b2.shape)],
        out_specs=pl.BlockSpec((BN, dout), lambda i: (i, 0)),
        out_shape=jax.ShapeDtypeStruct((n, dout), jnp.float32),
    )(x, w1, b1, w2, b2)


def _mlp_edges_body(x_ref, w1_ref, b1_ref, w2_ref, b2_ref, o0, o1, o2, o3):
    hdn = jnp.maximum(_dot(x_ref[...], w1_ref[...]) + b1_ref[...], 0.0)
    o = _dot(hdn, w2_ref[...]) + b2_ref[...]
    for g, oref in enumerate((o0, o1, o2, o3)):
        oref[...] = o[:, 16 * g:16 * (g + 1)]


def _mlp_edges(ea, w1, b1, w2, b2):
    din = ea.shape[1]
    grid = NEP // BE
    last = (NE + BE - 1) // BE - 1
    return pl.pallas_call(
        _mlp_edges_body,
        grid=(grid,),
        in_specs=[pl.BlockSpec((BE, din), lambda i: (jnp.minimum(i, last), 0)),
                  _full(w1.shape), _full(b1.shape),
                  _full(w2.shape), _full(b2.shape)],
        out_specs=[pl.BlockSpec((BE, 16), lambda i: (i, 0))] * 4,
        out_shape=[jax.ShapeDtypeStruct((NEP, 16), jnp.float32)] * 4,
    )(ea, w1, b1, w2, b2)


def _comb_body(h_ref, parts_ref, w1_ref, b1_ref, w2_ref, b2_ref, o_ref):
    c = _dot(h_ref[...], w1_ref[0:HID, :]) + b1_ref[...]
    for g in range(4):
        ag = parts_ref[0, g] + parts_ref[1, g]
        c = c + _dot(ag, w1_ref[HID + 16 * g:HID + 16 * (g + 1), :])
    o_ref[...] = _dot(jnp.maximum(c, 0.0), w2_ref[...]) + b2_ref[...]


def _comb(h, parts, w1, b1, w2, b2):
    grid = N_NODES // BN
    return pl.pallas_call(
        _comb_body,
        grid=(grid,),
        in_specs=[pl.BlockSpec((BN, HID), lambda i: (i, 0)),
                  pl.BlockSpec((2, 4, BN, 16), lambda i: (0, 0, i, 0)),
                  _full(w1.shape), _full(b1.shape),
                  _full(w2.shape), _full(b2.shape)],
        out_specs=pl.BlockSpec((BN, HID), lambda i: (i, 0)),
        out_shape=jax.ShapeDtypeStruct((N_NODES, HID), jnp.float32),
    )(h, parts, w1, b1, w2, b2)


def _colsum_body(h_ref, o_ref):
    @pl.when(pl.program_id(0) == 0)
    def _():
        o_ref[...] = jnp.zeros_like(o_ref)
    o_ref[...] += jnp.sum(h_ref[...], axis=0, keepdims=True)


def _colsum(h):
    grid = N_NODES // BN
    return pl.pallas_call(
        _colsum_body,
        grid=(grid,),
        in_specs=[pl.BlockSpec((BN, HID), lambda i: (i, 0))],
        out_specs=pl.BlockSpec((1, HID), lambda i: (0, 0)),
        out_shape=jax.ShapeDtypeStruct((1, HID), jnp.float32),
    )(h)


def _head_body(cs_ref, w1_ref, b1_ref, w2_ref, b2_ref,
               w3_ref, b3_ref, w4_ref, b4_ref, o_ref):
    pooled = cs_ref[...] * (1.0 / N_NODES)
    r = jnp.maximum(_dot(pooled, w1_ref[...]) + b1_ref[...], 0.0)
    r = _dot(r, w2_ref[...]) + b2_ref[...]
    o = jnp.maximum(_dot(r, w3_ref[...]) + b3_ref[...], 0.0)
    o_ref[...] = _dot(o, w4_ref[...]) + b4_ref[...]


def _head(cs, w1, b1, w2, b2, w3, b3, w4, b4):
    args = (cs, w1, b1, w2, b2, w3, b3, w4, b4)
    return pl.pallas_call(
        _head_body,
        in_specs=[_full(a.shape) for a in args],
        out_specs=_full((1, 7)),
        out_shape=jax.ShapeDtypeStruct((1, 7), jnp.float32),
    )(*args)


def _sc_body(zr_h, msg_h, src_h, dst_h, ee0_h, ee1_h, ee2_h, ee3_h, out_h,
             zbuf, cbuf, srcb, idxb, dstb, rows, eeb, aggr, gsem):
    c = lax.axis_index("c")
    s = lax.axis_index("s")
    tile = c * 16 + s
    base_e = tile * PER_TILE
    pltpu.sync_copy(zr_h, zbuf)

    for g, ee_h in enumerate((ee0_h, ee1_h, ee2_h, ee3_h)):
        def zbody(i, carry):
            pltpu.sync_copy(zbuf, aggr.at[pl.ds(s * 6256 + i * ZCH, ZCH)])
            return carry
        lax.fori_loop(0, ZIT, zbody, 0)
        plsc.subcore_barrier()

        def ebody(i, carry):
            e = pl.multiple_of(base_e + i * CHUNK, CHUNK)
            pltpu.sync_copy(src_h.at[pl.ds(e, CHUNK)], srcb)
            for j in range(4):
                for k in range(8):
                    sl = pl.ds(j * 128 + k * 16, 16)
                    idxb[j, pl.ds(k * 16, 16)] = srcb[sl] * 4 + g
            cps = [pltpu.async_copy(msg_h.at[idxb.at[j]],
                                    rows.at[pl.ds(j * 128, 128)], gsem)
                   for j in range(4)]
            e128 = pl.multiple_of(tile * (PER_TILE // 128) + i * (CHUNK // 128), 4)
            pltpu.sync_copy(dst_h.at[pl.ds(e128, 4)], dstb)
            pltpu.sync_copy(ee_h.at[pl.ds(e, CHUNK)], eeb)
            for cp in cps:
                cp.wait()

            def mbody(j, carry):
                for jj in range(8):
                    r = j * 8 + jj
                    rows[r] = rows[r] * eeb[r]
                return carry
            lax.fori_loop(0, CHUNK // 8, mbody, 0)

            for j in range(4):
                pltpu.sync_copy(rows.at[pl.ds(j * 128, 128)],
                                aggr.at[dstb.at[j]], add=True)
            return carry
        lax.fori_loop(0, NCHUNK, ebody, 0)
        plsc.subcore_barrier()

        def dbody(i, carry):
            r0 = s * 6256 + i * ZCH
            pltpu.sync_copy(aggr.at[pl.ds(r0, ZCH)], cbuf)
            pltpu.sync_copy(cbuf, out_h.at[c, g, pl.ds(r0, ZCH)])
            return carry
        lax.fori_loop(0, ZIT, dbody, 0)
        plsc.subcore_barrier()


def _sc_gms(zrows, msg_flat, src_p, dst2d, ee):
    mesh = plsc.VectorSubcoreMesh(core_axis_name="c", subcore_axis_name="s")
    f = functools.partial(
        pl.kernel, mesh=mesh,
        compiler_params=pltpu.CompilerParams(use_tc_tiling_on_sc=False),
        out_type=jax.ShapeDtypeStruct((2, 4, AGG_ROWS, 16), jnp.float32),
        scratch_types=[
            pltpu.VMEM((ZCH, 16), jnp.float32),
            pltpu.VMEM((ZCH, 16), jnp.float32),
            pltpu.VMEM((CHUNK,), jnp.int32),
            pltpu.VMEM((4, 128), jnp.int32),
            pltpu.VMEM((4, 128), jnp.int32),
            pltpu.VMEM((CHUNK, 16), jnp.float32),
            pltpu.VMEM((CHUNK, 16), jnp.float32),
            pltpu.VMEM_SHARED((AGG_ROWS, 16), jnp.float32),
            pltpu.SemaphoreType.DMA,
        ],
    )(_sc_body)
    return f(zrows, msg_flat, src_p, dst2d, ee[0], ee[1], ee[2], ee[3])


def kernel(x, edge_index, edge_attr, params):
    p = params
    ei = edge_index.astype(jnp.int32)
    pad = NEP - NE
    src_p = jnp.concatenate([ei[0], jnp.zeros((pad,), jnp.int32)])
    dst2d = jnp.concatenate([ei[1], jnp.full((pad,), JUNK, jnp.int32)])
    dst2d = dst2d.reshape(NEP // 128, 128)
    zrows = jnp.zeros((ZCH, 16), jnp.float32)

    h = _lin(x, p['node_emb_W'], p['node_emb_b'][None, :])
    for l in range(3):
        wf = p['edge_emb_W'] @ p[f'l{l}_edge_W1']
        bf = p['edge_emb_b'] @ p[f'l{l}_edge_W1'] + p[f'l{l}_edge_b1']
        ee = _mlp_edges(edge_attr, wf, bf[None, :],
                        p[f'l{l}_edge_W2'], p[f'l{l}_edge_b2'][None, :])
        msg = _mlp_nodes(h, p[f'l{l}_node_W1'], p[f'l{l}_node_b1'][None, :],
                         p[f'l{l}_node_W2'], p[f'l{l}_node_b2'][None, :])
        parts = _sc_gms(zrows, msg.reshape(4 * N_NODES, 16), src_p, dst2d, ee)
        h = _comb(h, parts, p[f'l{l}_comb_W1'], p[f'l{l}_comb_b1'][None, :],
                  p[f'l{l}_comb_W2'], p[f'l{l}_comb_b2'][None, :])
    cs = _colsum(h)
    return _head(cs, p['ro_W1'], p['ro_b1'][None, :],
                 p['ro_W2'], p['ro_b2'][None, :],
                 p['out_W1'], p['out_b1'][None, :],
                 p['out_W2'], p['out_b2'][None, :])


# speedup vs baseline: 1.7364x; 1.0089x over previous
import functools

import jax
import jax.numpy as jnp
from jax import lax
from jax.experimental import pallas as pl
from jax.experimental.pallas import tpu as pltpu
from jax.experimental.pallas import tpu_sc as plsc

N_NODES = 100000
HID = 64
NE = 1600000

CHUNK = 256
NCHUNK = 196
PER_TILE = CHUNK * NCHUNK
NEP = PER_TILE * 32
JUNK = N_NODES
AGG_ROWS = 100096
ZCH = 136
ZIT = 46

BN = 2000
BE = 2048


def _dot(a, b):
    return jnp.dot(a, b, preferred_element_type=jnp.float32)


def _lin_body(x_ref, w_ref, b_ref, o_ref):
    o_ref[...] = _dot(x_ref[...], w_ref[...]) + b_ref[...]


def _mlp_body(x_ref, w1_ref, b1_ref, w2_ref, b2_ref, o_ref):
    hdn = jnp.maximum(_dot(x_ref[...], w1_ref[...]) + b1_ref[...], 0.0)
    o_ref[...] = _dot(hdn, w2_ref[...]) + b2_ref[...]


def _full(shape):
    return pl.BlockSpec(shape, lambda *i: (0,) * len(shape))


def _lin(x, w, b):
    n, din = x.shape
    dout = w.shape[1]
    grid = n // BN
    return pl.pallas_call(
        _lin_body,
        grid=(grid,),
        in_specs=[pl.BlockSpec((BN, din), lambda i: (i, 0)),
                  _full(w.shape), _full(b.shape)],
        out_specs=pl.BlockSpec((BN, dout), lambda i: (i, 0)),
        out_shape=jax.ShapeDtypeStruct((n, dout), jnp.float32),
    )(x, w, b)


def _mlp_nodes(x, w1, b1, w2, b2):
    n, din = x.shape
    dout = w2.shape[1]
    grid = n // BN
    return pl.pallas_call(
        _mlp_body,
        grid=(grid,),
        in_specs=[pl.BlockSpec((BN, din), lambda i: (i, 0)),
                  _full(w1.shape), _full(b1.shape),
                  _full(w2.shape), _full(b2.shape)],
        out_specs=pl.BlockSpec((BN, dout), lambda i: (i, 0)),
        out_shape=jax.ShapeDtypeStruct((n, dout), jnp.float32),
    )(x, w1, b1, w2, b2)


def _mlp_edges_body(x_ref, w1_ref, b1_ref, w2_ref, b2_ref, o0, o1, o2, o3):
    hdn = jnp.maximum(_dot(x_ref[...], w1_ref[...]) + b1_ref[...], 0.0)
    o = _dot(hdn, w2_ref[...]) + b2_ref[...]
    for g, oref in enumerate((o0, o1, o2, o3)):
        oref[...] = o[:, 16 * g:16 * (g + 1)]


def _mlp_edges(ea, w1, b1, w2, b2):
    din = ea.shape[1]
    grid = NEP // BE
    last = (NE + BE - 1) // BE - 1
    return pl.pallas_call(
        _mlp_edges_body,
        grid=(grid,),
        in_specs=[pl.BlockSpec((BE, din), lambda i: (jnp.minimum(i, last), 0)),
                  _full(w1.shape), _full(b1.shape),
                  _full(w2.shape), _full(b2.shape)],
        out_specs=[pl.BlockSpec((BE, 16), lambda i: (i, 0))] * 4,
        out_shape=[jax.ShapeDtypeStruct((NEP, 16), jnp.float32)] * 4,
    )(ea, w1, b1, w2, b2)


def _comb_body(h_ref, parts_ref, w1_ref, b1_ref, w2_ref, b2_ref, o_ref):
    c = _dot(h_ref[...], w1_ref[0:HID, :]) + b1_ref[...]
    for g in range(4):
        ag = parts_ref[0, g] + parts_ref[1, g]
        c = c + _dot(ag, w1_ref[HID + 16 * g:HID + 16 * (g + 1), :])
    o_ref[...] = _dot(jnp.maximum(c, 0.0), w2_ref[...]) + b2_ref[...]


def _comb(h, parts, w1, b1, w2, b2):
    grid = N_NODES // BN
    return pl.pallas_call(
        _comb_body,
        grid=(grid,),
        in_specs=[pl.BlockSpec((BN, HID), lambda i: (i, 0)),
                  pl.BlockSpec((2, 4, BN, 16), lambda i: (0, 0, i, 0)),
                  _full(w1.shape), _full(b1.shape),
                  _full(w2.shape), _full(b2.shape)],
        out_specs=pl.BlockSpec((BN, HID), lambda i: (i, 0)),
        out_shape=jax.ShapeDtypeStruct((N_NODES, HID), jnp.float32),
    )(h, parts, w1, b1, w2, b2)


def _colsum_body(h_ref, o_ref):
    @pl.when(pl.program_id(0) == 0)
    def _():
        o_ref[...] = jnp.zeros_like(o_ref)
    o_ref[...] += jnp.sum(h_ref[...], axis=0, keepdims=True)


def _colsum(h):
    grid = N_NODES // BN
    return pl.pallas_call(
        _colsum_body,
        grid=(grid,),
        in_specs=[pl.BlockSpec((BN, HID), lambda i: (i, 0))],
        out_specs=pl.BlockSpec((1, HID), lambda i: (0, 0)),
        out_shape=jax.ShapeDtypeStruct((1, HID), jnp.float32),
    )(h)


def _head_body(cs_ref, w1_ref, b1_ref, w2_ref, b2_ref,
               w3_ref, b3_ref, w4_ref, b4_ref, o_ref):
    pooled = cs_ref[...] * (1.0 / N_NODES)
    r = jnp.maximum(_dot(pooled, w1_ref[...]) + b1_ref[...], 0.0)
    r = _dot(r, w2_ref[...]) + b2_ref[...]
    o = jnp.maximum(_dot(r, w3_ref[...]) + b3_ref[...], 0.0)
    o_ref[...] = _dot(o, w4_ref[...]) + b4_ref[...]


def _head(cs, w1, b1, w2, b2, w3, b3, w4, b4):
    args = (cs, w1, b1, w2, b2, w3, b3, w4, b4)
    return pl.pallas_call(
        _head_body,
        in_specs=[_full(a.shape) for a in args],
        out_specs=_full((1, 7)),
        out_shape=jax.ShapeDtypeStruct((1, 7), jnp.float32),
    )(*args)


def _sc_body(zr_h, msg_h, src_h, dst_h, ee0_h, ee1_h, ee2_h, ee3_h, out_h,
             zbuf, cbuf,
             srcb0, dstb0, idxb0, rows0, eeb0,
             srcb1, dstb1, idxb1, rows1, eeb1,
             aggr, spre0, sga0, sgb0, ssc0, spre1, sga1, sgb1, ssc1):
    c = lax.axis_index("c")
    s = lax.axis_index("s")
    tile = c * 16 + s
    pltpu.sync_copy(zr_h, zbuf)
    slots = ((srcb0, dstb0, idxb0, rows0, eeb0, spre0, (sga0, sgb0), ssc0),
             (srcb1, dstb1, idxb1, rows1, eeb1, spre1, (sga1, sgb1), ssc1))

    def loads(ee_h, k, slot):
        srcb, dstb, idxb, rows, eeb, spre, sgs, ssc = slot
        e = pl.multiple_of(tile * PER_TILE + k * CHUNK, CHUNK)
        e128 = pl.multiple_of(tile * (PER_TILE // 128) + k * (CHUNK // 128),
                              CHUNK // 128)
        return (pltpu.make_async_copy(src_h.at[pl.ds(e, CHUNK)], srcb, spre),
                pltpu.make_async_copy(dst_h.at[pl.ds(e128, CHUNK // 128)],
                                      dstb, spre),
                pltpu.make_async_copy(ee_h.at[pl.ds(e, CHUNK)], eeb, spre))

    def gathers(slot):
        srcb, dstb, idxb, rows, eeb, spre, sgs, ssc = slot
        return [pltpu.make_async_copy(msg_h.at[idxb.at[j]],
                                      rows.at[pl.ds(j * 128, 128)], sgs[j])
                for j in range(CHUNK // 128)]

    def scatters(slot):
        srcb, dstb, idxb, rows, eeb, spre, sgs, ssc = slot
        return [pltpu.make_async_copy(rows.at[pl.ds(j * 128, 128)],
                                      aggr.at[dstb.at[j]], ssc)
                for j in range(CHUNK // 128)]

    for g, ee_h in enumerate((ee0_h, ee1_h, ee2_h, ee3_h)):
        def zbody(i, carry):
            pltpu.sync_copy(zbuf, aggr.at[pl.ds(s * 6256 + i * ZCH, ZCH)])
            return carry
        lax.fori_loop(0, ZIT, zbody, 0)
        plsc.subcore_barrier()

        for cp in loads(ee_h, 0, slots[0]):
            cp.start()

        def pair(i2, carry):
            for half in range(2):
                k = i2 * 2 + half
                cur = slots[half]
                nxt = slots[1 - half]
                srcb, dstb, idxb, rows, eeb, spre, sgs, ssc = cur
                if half == 1:
                    for d in scatters(nxt):
                        d.wait()
                else:
                    @pl.when(i2 > 0)
                    def _():
                        for d in scatters(nxt):
                            d.wait()
                if half == 0:
                    for cp in loads(ee_h, k + 1, nxt):
                        cp.start()
                else:
                    @pl.when(k + 1 < NCHUNK)
                    def _():
                        for cp in loads(ee_h, k + 1, nxt):
                            cp.start()
                for cp in loads(ee_h, k, cur):
                    cp.wait()
                gds = gathers(cur)
                for j in range(CHUNK // 128):
                    for kk in range(8):
                        sl = pl.ds(j * 128 + kk * 16, 16)
                        idxb[j, pl.ds(kk * 16, 16)] = srcb[sl] * 4 + g
                    gds[j].start()
                for j in range(CHUNK // 128):
                    gds[j].wait()

                    def mbody(jj, cc, j=j):
                        for q in range(8):
                            r = j * 128 + jj * 8 + q
                            rows[r] = rows[r] * eeb[r]
                        return cc
                    lax.fori_loop(0, 16, mbody, 0)
                for j in range(CHUNK // 128):
                    pltpu.async_copy(rows.at[pl.ds(j * 128, 128)],
                                     aggr.at[dstb.at[j]], ssc, add=True)
            return carry
        lax.fori_loop(0, NCHUNK // 2, pair, 0)
        for d in scatters(slots[1]):
            d.wait()
        plsc.subcore_barrier()

        def dbody(i, carry):
            r0 = s * 6256 + i * ZCH
            pltpu.sync_copy(aggr.at[pl.ds(r0, ZCH)], cbuf)
            pltpu.sync_copy(cbuf, out_h.at[c, g, pl.ds(r0, ZCH)])
            return carry
        lax.fori_loop(0, ZIT, dbody, 0)
        plsc.subcore_barrier()


def _sc_gms(zrows, msg_flat, src_p, dst2d, ee):
    mesh = plsc.VectorSubcoreMesh(core_axis_name="c", subcore_axis_name="s")
    f = functools.partial(
        pl.kernel, mesh=mesh,
        compiler_params=pltpu.CompilerParams(use_tc_tiling_on_sc=False),
        out_type=jax.ShapeDtypeStruct((2, 4, AGG_ROWS, 16), jnp.float32),
        scratch_types=[
            pltpu.VMEM((ZCH, 16), jnp.float32),
            pltpu.VMEM((ZCH, 16), jnp.float32),
        ] + 2 * [
            pltpu.VMEM((CHUNK,), jnp.int32),
            pltpu.VMEM((CHUNK // 128, 128), jnp.int32),
            pltpu.VMEM((CHUNK // 128, 128), jnp.int32),
            pltpu.VMEM((CHUNK, 16), jnp.float32),
            pltpu.VMEM((CHUNK, 16), jnp.float32),
        ] + [
            pltpu.VMEM_SHARED((AGG_ROWS, 16), jnp.float32),
        ] + 8 * [pltpu.SemaphoreType.DMA],
    )(_sc_body)
    return f(zrows, msg_flat, src_p, dst2d, ee[0], ee[1], ee[2], ee[3])


def kernel(x, edge_index, edge_attr, params):
    p = params
    ei = edge_index.astype(jnp.int32)
    pad = NEP - NE
    src_p = jnp.concatenate([ei[0], jnp.zeros((pad,), jnp.int32)])
    dst2d = jnp.concatenate([ei[1], jnp.full((pad,), JUNK, jnp.int32)])
    dst2d = dst2d.reshape(NEP // 128, 128)
    zrows = jnp.zeros((ZCH, 16), jnp.float32)

    h = _lin(x, p['node_emb_W'], p['node_emb_b'][None, :])
    for l in range(3):
        wf = p['edge_emb_W'] @ p[f'l{l}_edge_W1']
        bf = p['edge_emb_b'] @ p[f'l{l}_edge_W1'] + p[f'l{l}_edge_b1']
        ee = _mlp_edges(edge_attr, wf, bf[None, :],
                        p[f'l{l}_edge_W2'], p[f'l{l}_edge_b2'][None, :])
        msg = _mlp_nodes(h, p[f'l{l}_node_W1'], p[f'l{l}_node_b1'][None, :],
                         p[f'l{l}_node_W2'], p[f'l{l}_node_b2'][None, :])
        parts = _sc_gms(zrows, msg.reshape(4 * N_NODES, 16), src_p, dst2d, ee)
        h = _comb(h, parts, p[f'l{l}_comb_W1'], p[f'l{l}_comb_b1'][None, :],
                  p[f'l{l}_comb_W2'], p[f'l{l}_comb_b2'][None, :])
    cs = _colsum(h)
    return _head(cs, p['ro_W1'], p['ro_b1'][None, :],
                 p['ro_W2'], p['ro_b2'][None, :],
                 p['out_W1'], p['out_b1'][None, :],
                 p['out_W2'], p['out_b2'][None, :])


# speedup vs baseline: 3.0277x; 1.7436x over previous
import functools

import jax
import jax.numpy as jnp
from jax import lax
from jax.experimental import pallas as pl
from jax.experimental.pallas import tpu as pltpu
from jax.experimental.pallas import tpu_sc as plsc

N_NODES = 100000
HID = 64
NE = 1600000

CHUNK = 256
NCHUNK = 196
PER_TILE = CHUNK * NCHUNK
NEP = PER_TILE * 32
JUNK = N_NODES
AGG_ROWS = 100096
ZCH = 136
ZIT = 46

BN = 2000
BE = 2048


def _dot(a, b):
    return jnp.dot(a, b, preferred_element_type=jnp.float32)


def _lin_body(x_ref, w_ref, b_ref, o_ref):
    o_ref[...] = _dot(x_ref[...], w_ref[...]) + b_ref[...]


def _mlp_body(x_ref, w1_ref, b1_ref, w2_ref, b2_ref, o_ref):
    hdn = jnp.maximum(_dot(x_ref[...], w1_ref[...]) + b1_ref[...], 0.0)
    o_ref[...] = _dot(hdn, w2_ref[...]) + b2_ref[...]


def _full(shape):
    return pl.BlockSpec(shape, lambda *i: (0,) * len(shape))


def _lin(x, w, b):
    n, din = x.shape
    dout = w.shape[1]
    grid = n // BN
    return pl.pallas_call(
        _lin_body,
        grid=(grid,),
        in_specs=[pl.BlockSpec((BN, din), lambda i: (i, 0)),
                  _full(w.shape), _full(b.shape)],
        out_specs=pl.BlockSpec((BN, dout), lambda i: (i, 0)),
        out_shape=jax.ShapeDtypeStruct((n, dout), jnp.float32),
    )(x, w, b)


def _mlp_nodes(x, w1, b1, w2, b2):
    n, din = x.shape
    dout = w2.shape[1]
    grid = n // BN
    return pl.pallas_call(
        _mlp_body,
        grid=(grid,),
        in_specs=[pl.BlockSpec((BN, din), lambda i: (i, 0)),
                  _full(w1.shape), _full(b1.shape),
                  _full(w2.shape), _full(b2.shape)],
        out_specs=pl.BlockSpec((BN, dout), lambda i: (i, 0)),
        out_shape=jax.ShapeDtypeStruct((n, dout), jnp.float32),
    )(x, w1, b1, w2, b2)


def _mlp_edges(ea, w1, b1, w2, b2):
    din = ea.shape[1]
    grid = NEP // BE
    last = (NE + BE - 1) // BE - 1
    return pl.pallas_call(
        _mlp_body,
        grid=(grid,),
        in_specs=[pl.BlockSpec((BE, din), lambda i: (jnp.minimum(i, last), 0)),
                  _full(w1.shape), _full(b1.shape),
                  _full(w2.shape), _full(b2.shape)],
        out_specs=pl.BlockSpec((BE, HID), lambda i: (i, 0)),
        out_shape=jax.ShapeDtypeStruct((NEP, HID), jnp.float32),
    )(ea, w1, b1, w2, b2)


def _comb_body(h_ref, parts_ref, w1_ref, b1_ref, w2_ref, b2_ref, o_ref):
    c = _dot(h_ref[...], w1_ref[0:HID, :]) + b1_ref[...]
    for g in range(4):
        ag = parts_ref[0, g] + parts_ref[1, g]
        c = c + _dot(ag, w1_ref[HID + 16 * g:HID + 16 * (g + 1), :])
    o_ref[...] = _dot(jnp.maximum(c, 0.0), w2_ref[...]) + b2_ref[...]


def _comb(h, parts, w1, b1, w2, b2):
    grid = N_NODES // BN
    return pl.pallas_call(
        _comb_body,
        grid=(grid,),
        in_specs=[pl.BlockSpec((BN, HID), lambda i: (i, 0)),
                  pl.BlockSpec((2, 4, BN, 16), lambda i: (0, 0, i, 0)),
                  _full(w1.shape), _full(b1.shape),
                  _full(w2.shape), _full(b2.shape)],
        out_specs=pl.BlockSpec((BN, HID), lambda i: (i, 0)),
        out_shape=jax.ShapeDtypeStruct((N_NODES, HID), jnp.float32),
    )(h, parts, w1, b1, w2, b2)


def _colsum_body(h_ref, o_ref):
    @pl.when(pl.program_id(0) == 0)
    def _():
        o_ref[...] = jnp.zeros_like(o_ref)
    o_ref[...] += jnp.sum(h_ref[...], axis=0, keepdims=True)


def _colsum(h):
    grid = N_NODES // BN
    return pl.pallas_call(
        _colsum_body,
        grid=(grid,),
        in_specs=[pl.BlockSpec((BN, HID), lambda i: (i, 0))],
        out_specs=pl.BlockSpec((1, HID), lambda i: (0, 0)),
        out_shape=jax.ShapeDtypeStruct((1, HID), jnp.float32),
    )(h)


def _head_body(cs_ref, w1_ref, b1_ref, w2_ref, b2_ref,
               w3_ref, b3_ref, w4_ref, b4_ref, o_ref):
    pooled = cs_ref[...] * (1.0 / N_NODES)
    r = jnp.maximum(_dot(pooled, w1_ref[...]) + b1_ref[...], 0.0)
    r = _dot(r, w2_ref[...]) + b2_ref[...]
    o = jnp.maximum(_dot(r, w3_ref[...]) + b3_ref[...], 0.0)
    o_ref[...] = _dot(o, w4_ref[...]) + b4_ref[...]


def _head(cs, w1, b1, w2, b2, w3, b3, w4, b4):
    args = (cs, w1, b1, w2, b2, w3, b3, w4, b4)
    return pl.pallas_call(
        _head_body,
        in_specs=[_full(a.shape) for a in args],
        out_specs=_full((1, 7)),
        out_shape=jax.ShapeDtypeStruct((1, 7), jnp.float32),
    )(*args)


def _sc_body(zr_h, msg_h, src_h, dst_h, ee_h, out_h,
             zbuf, cbuf,
             srcb0, dstb0, idxb0, rows0, eeb0,
             srcb1, dstb1, idxb1, rows1, eeb1,
             aggr, spre0, sga0, sgb0, ssc0, spre1, sga1, sgb1, ssc1):
    c = lax.axis_index("c")
    s = lax.axis_index("s")
    tile = c * 16 + s
    pltpu.sync_copy(zr_h, zbuf)
    slots = ((srcb0, dstb0, idxb0, rows0, eeb0, spre0, (sga0, sgb0), ssc0),
             (srcb1, dstb1, idxb1, rows1, eeb1, spre1, (sga1, sgb1), ssc1))

    def loads(g, k, slot):
        srcb, dstb, idxb, rows, eeb, spre, sgs, ssc = slot
        e = pl.multiple_of(tile * PER_TILE + k * CHUNK, CHUNK)
        e128 = pl.multiple_of(tile * (PER_TILE // 128) + k * (CHUNK // 128),
                              CHUNK // 128)
        return (pltpu.make_async_copy(src_h.at[pl.ds(e, CHUNK)], srcb, spre),
                pltpu.make_async_copy(dst_h.at[pl.ds(e128, CHUNK // 128)],
                                      dstb, spre),
                pltpu.make_async_copy(
                    ee_h.at[pl.ds(e, CHUNK), pl.ds(g * 16, 16)], eeb, spre))

    def gathers(slot):
        srcb, dstb, idxb, rows, eeb, spre, sgs, ssc = slot
        return [pltpu.make_async_copy(msg_h.at[idxb.at[j]],
                                      rows.at[pl.ds(j * 128, 128)], sgs[j])
                for j in range(CHUNK // 128)]

    def scatters(slot):
        srcb, dstb, idxb, rows, eeb, spre, sgs, ssc = slot
        return [pltpu.make_async_copy(rows.at[pl.ds(j * 128, 128)],
                                      aggr.at[dstb.at[j]], ssc)
                for j in range(CHUNK // 128)]

    for g in range(4):
        def zbody(i, carry):
            pltpu.sync_copy(zbuf, aggr.at[pl.ds(s * 6256 + i * ZCH, ZCH)])
            return carry
        lax.fori_loop(0, ZIT, zbody, 0)
        plsc.subcore_barrier()

        for cp in loads(g, 0, slots[0]):
            cp.start()

        def pair(i2, carry):
            for half in range(2):
                k = i2 * 2 + half
                cur = slots[half]
                nxt = slots[1 - half]
                srcb, dstb, idxb, rows, eeb, spre, sgs, ssc = cur
                if half == 1:
                    for d in scatters(nxt):
                        d.wait()
                else:
                    @pl.when(i2 > 0)
                    def _():
                        for d in scatters(nxt):
                            d.wait()
                if half == 0:
                    for cp in loads(g, k + 1, nxt):
                        cp.start()
                else:
                    @pl.when(k + 1 < NCHUNK)
                    def _():
                        for cp in loads(g, k + 1, nxt):
                            cp.start()
                for cp in loads(g, k, cur):
                    cp.wait()
                gds = gathers(cur)
                for j in range(CHUNK // 128):
                    for kk in range(8):
                        sl = pl.ds(j * 128 + kk * 16, 16)
                        idxb[j, pl.ds(kk * 16, 16)] = srcb[sl] * 4 + g
                    gds[j].start()
                for j in range(CHUNK // 128):
                    gds[j].wait()

                    def mbody(jj, cc, j=j):
                        for q in range(8):
                            r = j * 128 + jj * 8 + q
                            rows[r] = rows[r] * eeb[r]
                        return cc
                    lax.fori_loop(0, 16, mbody, 0)
                for j in range(CHUNK // 128):
                    pltpu.async_copy(rows.at[pl.ds(j * 128, 128)],
                                     aggr.at[dstb.at[j]], ssc, add=True)
            return carry
        lax.fori_loop(0, NCHUNK // 2, pair, 0)
        for d in scatters(slots[1]):
            d.wait()
        plsc.subcore_barrier()

        def dbody(i, carry):
            r0 = s * 6256 + i * ZCH
            pltpu.sync_copy(aggr.at[pl.ds(r0, ZCH)], cbuf)
            pltpu.sync_copy(cbuf, out_h.at[c, g, pl.ds(r0, ZCH)])
            return carry
        lax.fori_loop(0, ZIT, dbody, 0)
        plsc.subcore_barrier()


def _sc_gms(zrows, msg_flat, src_p, dst2d, ee):
    mesh = plsc.VectorSubcoreMesh(core_axis_name="c", subcore_axis_name="s")
    f = functools.partial(
        pl.kernel, mesh=mesh,
        compiler_params=pltpu.CompilerParams(use_tc_tiling_on_sc=False),
        out_type=jax.ShapeDtypeStruct((2, 4, AGG_ROWS, 16), jnp.float32),
        scratch_types=[
            pltpu.VMEM((ZCH, 16), jnp.float32),
            pltpu.VMEM((ZCH, 16), jnp.float32),
        ] + 2 * [
            pltpu.VMEM((CHUNK,), jnp.int32),
            pltpu.VMEM((CHUNK // 128, 128), jnp.int32),
            pltpu.VMEM((CHUNK // 128, 128), jnp.int32),
            pltpu.VMEM((CHUNK, 16), jnp.float32),
            pltpu.VMEM((CHUNK, 16), jnp.float32),
        ] + [
            pltpu.VMEM_SHARED((AGG_ROWS, 16), jnp.float32),
        ] + 8 * [pltpu.SemaphoreType.DMA],
    )(_sc_body)
    return f(zrows, msg_flat, src_p, dst2d, ee)


def kernel(x, edge_index, edge_attr, params):
    p = params
    ei = edge_index.astype(jnp.int32)
    pad = NEP - NE
    src_p = jnp.concatenate([ei[0], jnp.zeros((pad,), jnp.int32)])
    dst2d = jnp.concatenate([ei[1], jnp.full((pad,), JUNK, jnp.int32)])
    dst2d = dst2d.reshape(NEP // 128, 128)
    zrows = jnp.zeros((ZCH, 16), jnp.float32)

    h = _lin(x, p['node_emb_W'], p['node_emb_b'][None, :])
    for l in range(3):
        wf = p['edge_emb_W'] @ p[f'l{l}_edge_W1']
        bf = p['edge_emb_b'] @ p[f'l{l}_edge_W1'] + p[f'l{l}_edge_b1']
        ee = _mlp_edges(edge_attr, wf, bf[None, :],
                        p[f'l{l}_edge_W2'], p[f'l{l}_edge_b2'][None, :])
        msg = _mlp_nodes(h, p[f'l{l}_node_W1'], p[f'l{l}_node_b1'][None, :],
                         p[f'l{l}_node_W2'], p[f'l{l}_node_b2'][None, :])
        parts = _sc_gms(zrows, msg.reshape(4 * N_NODES, 16), src_p, dst2d, ee)
        h = _comb(h, parts, p[f'l{l}_comb_W1'], p[f'l{l}_comb_b1'][None, :],
                  p[f'l{l}_comb_W2'], p[f'l{l}_comb_b2'][None, :])
    cs = _colsum(h)
    return _head(cs, p['ro_W1'], p['ro_b1'][None, :],
                 p['ro_W2'], p['ro_b2'][None, :],
                 p['out_W1'], p['out_b1'][None, :],
                 p['out_W2'], p['out_b2'][None, :])


# speedup vs baseline: 3.0379x; 1.0034x over previous
import functools

import jax
import jax.numpy as jnp
from jax import lax
from jax.experimental import pallas as pl
from jax.experimental.pallas import tpu as pltpu
from jax.experimental.pallas import tpu_sc as plsc

N_NODES = 100000
HID = 64
NE = 1600000

CHUNK = 256
NCHUNK = 196
PER_TILE = CHUNK * NCHUNK
NEP = PER_TILE * 32
JUNK = N_NODES
AGG_ROWS = 100096
TROWS = 6256

BN = 2000
BE = 2048


def _dot(a, b):
    return jnp.dot(a, b, preferred_element_type=jnp.float32)


def _lin_body(x_ref, w_ref, b_ref, o_ref):
    o_ref[...] = _dot(x_ref[...], w_ref[...]) + b_ref[...]


def _mlp_body(x_ref, w1_ref, b1_ref, w2_ref, b2_ref, o_ref):
    hdn = jnp.maximum(_dot(x_ref[...], w1_ref[...]) + b1_ref[...], 0.0)
    o_ref[...] = _dot(hdn, w2_ref[...]) + b2_ref[...]


def _full(shape):
    return pl.BlockSpec(shape, lambda *i: (0,) * len(shape))


def _lin(x, w, b):
    n, din = x.shape
    dout = w.shape[1]
    grid = n // BN
    return pl.pallas_call(
        _lin_body,
        grid=(grid,),
        in_specs=[pl.BlockSpec((BN, din), lambda i: (i, 0)),
                  _full(w.shape), _full(b.shape)],
        out_specs=pl.BlockSpec((BN, dout), lambda i: (i, 0)),
        out_shape=jax.ShapeDtypeStruct((n, dout), jnp.float32),
    )(x, w, b)


def _mlp_nodes(x, w1, b1, w2, b2):
    n, din = x.shape
    dout = w2.shape[1]
    grid = n // BN
    return pl.pallas_call(
        _mlp_body,
        grid=(grid,),
        in_specs=[pl.BlockSpec((BN, din), lambda i: (i, 0)),
                  _full(w1.shape), _full(b1.shape),
                  _full(w2.shape), _full(b2.shape)],
        out_specs=pl.BlockSpec((BN, dout), lambda i: (i, 0)),
        out_shape=jax.ShapeDtypeStruct((n, dout), jnp.float32),
    )(x, w1, b1, w2, b2)


def _mlp_edges(ea, w1, b1, w2, b2):
    din = ea.shape[1]
    grid = NEP // BE
    last = (NE + BE - 1) // BE - 1
    return pl.pallas_call(
        _mlp_body,
        grid=(grid,),
        in_specs=[pl.BlockSpec((BE, din), lambda i: (jnp.minimum(i, last), 0)),
                  _full(w1.shape), _full(b1.shape),
                  _full(w2.shape), _full(b2.shape)],
        out_specs=pl.BlockSpec((BE, HID), lambda i: (i, 0)),
        out_shape=jax.ShapeDtypeStruct((NEP, HID), jnp.float32),
    )(ea, w1, b1, w2, b2)


def _comb_body(h_ref, parts_ref, w1_ref, b1_ref, w2_ref, b2_ref, o_ref):
    c = _dot(h_ref[...], w1_ref[0:HID, :]) + b1_ref[...]
    for g in range(4):
        ag = parts_ref[0, g] + parts_ref[1, g]
        c = c + _dot(ag, w1_ref[HID + 16 * g:HID + 16 * (g + 1), :])
    o_ref[...] = _dot(jnp.maximum(c, 0.0), w2_ref[...]) + b2_ref[...]


def _comb(h, parts, w1, b1, w2, b2):
    grid = N_NODES // BN
    return pl.pallas_call(
        _comb_body,
        grid=(grid,),
        in_specs=[pl.BlockSpec((BN, HID), lambda i: (i, 0)),
                  pl.BlockSpec((2, 4, BN, 16), lambda i: (0, 0, i, 0)),
                  _full(w1.shape), _full(b1.shape),
                  _full(w2.shape), _full(b2.shape)],
        out_specs=pl.BlockSpec((BN, HID), lambda i: (i, 0)),
        out_shape=jax.ShapeDtypeStruct((N_NODES, HID), jnp.float32),
    )(h, parts, w1, b1, w2, b2)


def _colsum_body(h_ref, o_ref):
    @pl.when(pl.program_id(0) == 0)
    def _():
        o_ref[...] = jnp.zeros_like(o_ref)
    o_ref[...] += jnp.sum(h_ref[...], axis=0, keepdims=True)


def _colsum(h):
    grid = N_NODES // BN
    return pl.pallas_call(
        _colsum_body,
        grid=(grid,),
        in_specs=[pl.BlockSpec((BN, HID), lambda i: (i, 0))],
        out_specs=pl.BlockSpec((1, HID), lambda i: (0, 0)),
        out_shape=jax.ShapeDtypeStruct((1, HID), jnp.float32),
    )(h)


def _head_body(cs_ref, w1_ref, b1_ref, w2_ref, b2_ref,
               w3_ref, b3_ref, w4_ref, b4_ref, o_ref):
    pooled = cs_ref[...] * (1.0 / N_NODES)
    r = jnp.maximum(_dot(pooled, w1_ref[...]) + b1_ref[...], 0.0)
    r = _dot(r, w2_ref[...]) + b2_ref[...]
    o = jnp.maximum(_dot(r, w3_ref[...]) + b3_ref[...], 0.0)
    o_ref[...] = _dot(o, w4_ref[...]) + b4_ref[...]


def _head(cs, w1, b1, w2, b2, w3, b3, w4, b4):
    args = (cs, w1, b1, w2, b2, w3, b3, w4, b4)
    return pl.pallas_call(
        _head_body,
        in_specs=[_full(a.shape) for a in args],
        out_specs=_full((1, 7)),
        out_shape=jax.ShapeDtypeStruct((1, 7), jnp.float32),
    )(*args)


def _sc_body(zr_h, msg_h, src_h, dst_h, ee_h, out_h,
             srcb0, dstb0, idxb0, rows0, eeb0,
             srcb1, dstb1, idxb1, rows1, eeb1,
             aggr, spre0, sga0, sgb0, ssc0, spre1, sga1, sgb1, ssc1):
    c = lax.axis_index("c")
    s = lax.axis_index("s")
    tile = c * 16 + s
    slots = ((srcb0, dstb0, idxb0, rows0, eeb0, spre0, (sga0, sgb0), ssc0),
             (srcb1, dstb1, idxb1, rows1, eeb1, spre1, (sga1, sgb1), ssc1))

    def loads(g, k, slot):
        srcb, dstb, idxb, rows, eeb, spre, sgs, ssc = slot
        e = pl.multiple_of(tile * PER_TILE + k * CHUNK, CHUNK)
        e128 = pl.multiple_of(tile * (PER_TILE // 128) + k * (CHUNK // 128),
                              CHUNK // 128)
        return (pltpu.make_async_copy(src_h.at[pl.ds(e, CHUNK)], srcb, spre),
                pltpu.make_async_copy(dst_h.at[pl.ds(e128, CHUNK // 128)],
                                      dstb, spre),
                pltpu.make_async_copy(
                    ee_h.at[pl.ds(e, CHUNK), pl.ds(g * 16, 16)], eeb, spre))

    def gathers(slot):
        srcb, dstb, idxb, rows, eeb, spre, sgs, ssc = slot
        return [pltpu.make_async_copy(msg_h.at[idxb.at[j]],
                                      rows.at[pl.ds(j * 128, 128)], sgs[j])
                for j in range(CHUNK // 128)]

    def scatters(slot):
        srcb, dstb, idxb, rows, eeb, spre, sgs, ssc = slot
        return [pltpu.make_async_copy(rows.at[pl.ds(j * 128, 128)],
                                      aggr.at[dstb.at[j]], ssc)
                for j in range(CHUNK // 128)]

    for g in range(4):
        pltpu.sync_copy(zr_h, aggr.at[pl.ds(s * TROWS, TROWS)])
        plsc.subcore_barrier()

        for cp in loads(g, 0, slots[0]):
            cp.start()

        def pair(i2, carry):
            for half in range(2):
                k = i2 * 2 + half
                cur = slots[half]
                nxt = slots[1 - half]
                srcb, dstb, idxb, rows, eeb, spre, sgs, ssc = cur
                if half == 1:
                    for d in scatters(nxt):
                        d.wait()
                else:
                    @pl.when(i2 > 0)
                    def _():
                        for d in scatters(nxt):
                            d.wait()
                if half == 0:
                    for cp in loads(g, k + 1, nxt):
                        cp.start()
                else:
                    @pl.when(k + 1 < NCHUNK)
                    def _():
                        for cp in loads(g, k + 1, nxt):
                            cp.start()
                for cp in loads(g, k, cur):
                    cp.wait()
                gds = gathers(cur)
                for j in range(CHUNK // 128):
                    for kk in range(8):
                        sl = pl.ds(j * 128 + kk * 16, 16)
                        idxb[j, pl.ds(kk * 16, 16)] = srcb[sl] * 4 + g
                    gds[j].start()
                for j in range(CHUNK // 128):
                    gds[j].wait()

                    def mbody(jj, cc, j=j):
                        for q in range(8):
                            r = j * 128 + jj * 8 + q
                            rows[r] = rows[r] * eeb[r]
                        return cc
                    lax.fori_loop(0, 16, mbody, 0)
                for j in range(CHUNK // 128):
                    pltpu.async_copy(rows.at[pl.ds(j * 128, 128)],
                                     aggr.at[dstb.at[j]], ssc, add=True)
            return carry
        lax.fori_loop(0, NCHUNK // 2, pair, 0)
        for d in scatters(slots[1]):
            d.wait()
        plsc.subcore_barrier()

        pltpu.sync_copy(aggr.at[pl.ds(s * TROWS, TROWS)],
                        out_h.at[c, g, pl.ds(s * TROWS, TROWS)])
        plsc.subcore_barrier()


def _sc_gms(zrows, msg_flat, src_p, dst2d, ee):
    mesh = plsc.VectorSubcoreMesh(core_axis_name="c", subcore_axis_name="s")
    f = functools.partial(
        pl.kernel, mesh=mesh,
        compiler_params=pltpu.CompilerParams(use_tc_tiling_on_sc=False),
        out_type=jax.ShapeDtypeStruct((2, 4, AGG_ROWS, 16), jnp.float32),
        scratch_types=2 * [
            pltpu.VMEM((CHUNK,), jnp.int32),
            pltpu.VMEM((CHUNK // 128, 128), jnp.int32),
            pltpu.VMEM((CHUNK // 128, 128), jnp.int32),
            pltpu.VMEM((CHUNK, 16), jnp.float32),
            pltpu.VMEM((CHUNK, 16), jnp.float32),
        ] + [
            pltpu.VMEM_SHARED((AGG_ROWS, 16), jnp.float32),
        ] + 8 * [pltpu.SemaphoreType.DMA],
    )(_sc_body)
    return f(zrows, msg_flat, src_p, dst2d, ee)


def kernel(x, edge_index, edge_attr, params):
    p = params
    ei = edge_index.astype(jnp.int32)
    pad = NEP - NE
    src_p = jnp.concatenate([ei[0], jnp.zeros((pad,), jnp.int32)])
    dst2d = jnp.concatenate([ei[1], jnp.full((pad,), JUNK, jnp.int32)])
    dst2d = dst2d.reshape(NEP // 128, 128)
    zrows = jnp.zeros((TROWS, 16), jnp.float32)

    h = _lin(x, p['node_emb_W'], p['node_emb_b'][None, :])
    for l in range(3):
        wf = p['edge_emb_W'] @ p[f'l{l}_edge_W1']
        bf = p['edge_emb_b'] @ p[f'l{l}_edge_W1'] + p[f'l{l}_edge_b1']
        ee = _mlp_edges(edge_attr, wf, bf[None, :],
                        p[f'l{l}_edge_W2'], p[f'l{l}_edge_b2'][None, :])
        msg = _mlp_nodes(h, p[f'l{l}_node_W1'], p[f'l{l}_node_b1'][None, :],
                         p[f'l{l}_node_W2'], p[f'l{l}_node_b2'][None, :])
        parts = _sc_gms(zrows, msg.reshape(4 * N_NODES, 16), src_p, dst2d, ee)
        h = _comb(h, parts, p[f'l{l}_comb_W1'], p[f'l{l}_comb_b1'][None, :],
                  p[f'l{l}_comb_W2'], p[f'l{l}_comb_b2'][None, :])
    cs = _colsum(h)
    return _head(cs, p['ro_W1'], p['ro_b1'][None, :],
                 p['ro_W2'], p['ro_b2'][None, :],
                 p['out_W1'], p['out_b1'][None, :],
                 p['out_W2'], p['out_b2'][None, :])


# speedup vs baseline: 3.3030x; 1.0873x over previous
import functools

import jax
import jax.numpy as jnp
from jax import lax
from jax.experimental import pallas as pl
from jax.experimental.pallas import tpu as pltpu
from jax.experimental.pallas import tpu_sc as plsc

N_NODES = 100000
HID = 64
NE = 1600000

CHUNK = 256
NCHUNK = 392
PER_TILE = CHUNK * NCHUNK
NEP = PER_TILE * 16
JUNK = N_NODES
AGG_ROWS = 100096
TROWS = 6256

BN = 2000
BE = 2048


def _dot(a, b):
    return jnp.dot(a, b, preferred_element_type=jnp.float32)


def _lin_body(x_ref, w_ref, b_ref, o_ref):
    o_ref[...] = _dot(x_ref[...], w_ref[...]) + b_ref[...]


def _mlp_body(x_ref, w1_ref, b1_ref, w2_ref, b2_ref, o_ref):
    hdn = jnp.maximum(_dot(x_ref[...], w1_ref[...]) + b1_ref[...], 0.0)
    o_ref[...] = _dot(hdn, w2_ref[...]) + b2_ref[...]


def _full(shape):
    return pl.BlockSpec(shape, lambda *i: (0,) * len(shape))


def _lin(x, w, b):
    n, din = x.shape
    dout = w.shape[1]
    grid = n // BN
    return pl.pallas_call(
        _lin_body,
        grid=(grid,),
        in_specs=[pl.BlockSpec((BN, din), lambda i: (i, 0)),
                  _full(w.shape), _full(b.shape)],
        out_specs=pl.BlockSpec((BN, dout), lambda i: (i, 0)),
        out_shape=jax.ShapeDtypeStruct((n, dout), jnp.float32),
    )(x, w, b)


def _mlp_nodes(x, w1, b1, w2, b2):
    n, din = x.shape
    dout = w2.shape[1]
    grid = n // BN
    return pl.pallas_call(
        _mlp_body,
        grid=(grid,),
        in_specs=[pl.BlockSpec((BN, din), lambda i: (i, 0)),
                  _full(w1.shape), _full(b1.shape),
                  _full(w2.shape), _full(b2.shape)],
        out_specs=pl.BlockSpec((BN, dout), lambda i: (i, 0)),
        out_shape=jax.ShapeDtypeStruct((n, dout), jnp.float32),
    )(x, w1, b1, w2, b2)


def _mlp_edges(ea, w1, b1, w2, b2):
    din = ea.shape[1]
    grid = NEP // BE
    last = (NE + BE - 1) // BE - 1
    return pl.pallas_call(
        _mlp_body,
        grid=(grid,),
        in_specs=[pl.BlockSpec((BE, din), lambda i: (jnp.minimum(i, last), 0)),
                  _full(w1.shape), _full(b1.shape),
                  _full(w2.shape), _full(b2.shape)],
        out_specs=pl.BlockSpec((BE, HID), lambda i: (i, 0)),
        out_shape=jax.ShapeDtypeStruct((NEP, HID), jnp.float32),
    )(ea, w1, b1, w2, b2)


def _comb_body(h_ref, parts_ref, w1_ref, b1_ref, w2_ref, b2_ref, o_ref):
    c = _dot(h_ref[...], w1_ref[0:HID, :]) + b1_ref[...]
    for g in range(4):
        c = c + _dot(parts_ref[g], w1_ref[HID + 16 * g:HID + 16 * (g + 1), :])
    o_ref[...] = _dot(jnp.maximum(c, 0.0), w2_ref[...]) + b2_ref[...]


def _comb(h, parts, w1, b1, w2, b2):
    grid = N_NODES // BN
    return pl.pallas_call(
        _comb_body,
        grid=(grid,),
        in_specs=[pl.BlockSpec((BN, HID), lambda i: (i, 0)),
                  pl.BlockSpec((4, BN, 16), lambda i: (0, i, 0)),
                  _full(w1.shape), _full(b1.shape),
                  _full(w2.shape), _full(b2.shape)],
        out_specs=pl.BlockSpec((BN, HID), lambda i: (i, 0)),
        out_shape=jax.ShapeDtypeStruct((N_NODES, HID), jnp.float32),
    )(h, parts, w1, b1, w2, b2)


def _colsum_body(h_ref, o_ref):
    @pl.when(pl.program_id(0) == 0)
    def _():
        o_ref[...] = jnp.zeros_like(o_ref)
    o_ref[...] += jnp.sum(h_ref[...], axis=0, keepdims=True)


def _colsum(h):
    grid = N_NODES // BN
    return pl.pallas_call(
        _colsum_body,
        grid=(grid,),
        in_specs=[pl.BlockSpec((BN, HID), lambda i: (i, 0))],
        out_specs=pl.BlockSpec((1, HID), lambda i: (0, 0)),
        out_shape=jax.ShapeDtypeStruct((1, HID), jnp.float32),
    )(h)


def _head_body(cs_ref, w1_ref, b1_ref, w2_ref, b2_ref,
               w3_ref, b3_ref, w4_ref, b4_ref, o_ref):
    pooled = cs_ref[...] * (1.0 / N_NODES)
    r = jnp.maximum(_dot(pooled, w1_ref[...]) + b1_ref[...], 0.0)
    r = _dot(r, w2_ref[...]) + b2_ref[...]
    o = jnp.maximum(_dot(r, w3_ref[...]) + b3_ref[...], 0.0)
    o_ref[...] = _dot(o, w4_ref[...]) + b4_ref[...]


def _head(cs, w1, b1, w2, b2, w3, b3, w4, b4):
    args = (cs, w1, b1, w2, b2, w3, b3, w4, b4)
    return pl.pallas_call(
        _head_body,
        in_specs=[_full(a.shape) for a in args],
        out_specs=_full((1, 7)),
        out_shape=jax.ShapeDtypeStruct((1, 7), jnp.float32),
    )(*args)


def _sc_body(zr_h, msg_h, src_h, dst_h, ee_h, out_h,
             srcb0, idxb0, rows0, eeb0, srcb1, idxb1, rows1, eeb1,
             dstb0, dstb1, dstb2, dstb3,
             aggr, spre0, sga0, sgb0, ssc0, spre1, sga1, sgb1, ssc1):
    c = lax.axis_index("c")
    s = lax.axis_index("s")
    slots = ((srcb0, idxb0, rows0, eeb0, spre0, (sga0, sgb0), ssc0),
             (srcb1, idxb1, rows1, eeb1, spre1, (sga1, sgb1), ssc1))
    dstbs = (dstb0, dstb1, dstb2, dstb3)

    def loads(g, k, kmod):
        srcb, idxb, rows, eeb, spre, sgs, ssc = slots[kmod % 2]
        dstb = dstbs[kmod % 4]
        e = pl.multiple_of(s * PER_TILE + k * CHUNK, CHUNK)
        e128 = pl.multiple_of(s * (PER_TILE // 128) + k * (CHUNK // 128),
                              CHUNK // 128)
        goff = pl.multiple_of(g * 16, 16)
        return (pltpu.make_async_copy(src_h.at[pl.ds(e, CHUNK)], srcb, spre),
                pltpu.make_async_copy(dst_h.at[pl.ds(e128, CHUNK // 128)],
                                      dstb, spre),
                pltpu.make_async_copy(ee_h.at[pl.ds(e, CHUNK),
                                              pl.ds(goff, 16)], eeb, spre))

    def gathers(kmod):
        srcb, idxb, rows, eeb, spre, sgs, ssc = slots[kmod % 2]
        return [pltpu.make_async_copy(msg_h.at[idxb.at[j]],
                                      rows.at[pl.ds(j * 128, 128)], sgs[j])
                for j in range(CHUNK // 128)]

    def scatters(kmod):
        srcb, idxb, rows, eeb, spre, sgs, ssc = slots[kmod % 2]
        return [pltpu.make_async_copy(rows.at[pl.ds(j * 128, 128)],
                                      aggr.at[dstbs[kmod % 4].at[j]], ssc)
                for j in range(CHUNK // 128)]

    def idx_and_gather(g, k, kmod):
        srcb, idxb, rows, eeb, spre, sgs, ssc = slots[kmod % 2]
        for cp in loads(g, k, kmod):
            cp.wait()
        gds = gathers(kmod)
        for j in range(CHUNK // 128):
            for kk in range(8):
                idxb[j, pl.ds(kk * 16, 16)] = (
                    srcb[pl.ds(j * 128 + kk * 16, 16)] * 4 + g)
            gds[j].start()

    def mul_scatter(kmod):
        srcb, idxb, rows, eeb, spre, sgs, ssc = slots[kmod % 2]
        gds = gathers(kmod)
        for j in range(CHUNK // 128):
            gds[j].wait()

            def mbody(jj, cc, j=j):
                for q in range(8):
                    r = j * 128 + jj * 8 + q
                    rows[r] = rows[r] * eeb[r]
                return cc
            lax.fori_loop(0, 16, mbody, 0)
        for j in range(CHUNK // 128):
            pltpu.async_copy(rows.at[pl.ds(j * 128, 128)],
                             aggr.at[dstbs[kmod % 4].at[j]],
                             slots[kmod % 2][6], add=True)

    for gg in range(2):
        g = c * 2 + gg
        pltpu.sync_copy(zr_h, aggr.at[pl.ds(s * TROWS, TROWS)])
        plsc.subcore_barrier()

        for cp in loads(g, 0, 0):
            cp.start()
        for cp in loads(g, 1, 1):
            cp.start()
        idx_and_gather(g, 0, 0)

        def quad(i2, carry):
            for half in range(4):
                k = i2 * 4 + half
                if half > 0:
                    for d in scatters(half - 1):
                        d.wait()
                else:
                    @pl.when(k > 0)
                    def _():
                        for d in scatters(3):
                            d.wait()
                if half < 3:
                    idx_and_gather(g, k + 1, half + 1)
                else:
                    @pl.when(k + 1 < NCHUNK)
                    def _():
                        idx_and_gather(g, k + 1, half + 1)
                mul_scatter(half)
                if half < 2:
                    for cp in loads(g, k + 2, half + 2):
                        cp.start()
                else:
                    @pl.when(k + 2 < NCHUNK)
                    def _():
                        for cp in loads(g, k + 2, half + 2):
                            cp.start()
            return carry
        lax.fori_loop(0, NCHUNK // 4, quad, 0)
        for d in scatters(NCHUNK - 1):
            d.wait()
        plsc.subcore_barrier()

        pltpu.sync_copy(aggr.at[pl.ds(s * TROWS, TROWS)],
                        out_h.at[g, pl.ds(s * TROWS, TROWS)])
        plsc.subcore_barrier()


def _sc_gms(zrows, msg_flat, src_p, dst2d, ee):
    mesh = plsc.VectorSubcoreMesh(core_axis_name="c", subcore_axis_name="s")
    f = functools.partial(
        pl.kernel, mesh=mesh,
        compiler_params=pltpu.CompilerParams(use_tc_tiling_on_sc=False),
        out_type=jax.ShapeDtypeStruct((4, AGG_ROWS, 16), jnp.float32),
        scratch_types=2 * [
            pltpu.VMEM((CHUNK,), jnp.int32),
            pltpu.VMEM((CHUNK // 128, 128), jnp.int32),
            pltpu.VMEM((CHUNK, 16), jnp.float32),
            pltpu.VMEM((CHUNK, 16), jnp.float32),
        ] + 4 * [
            pltpu.VMEM((CHUNK // 128, 128), jnp.int32),
        ] + [
            pltpu.VMEM_SHARED((AGG_ROWS, 16), jnp.float32),
        ] + 8 * [pltpu.SemaphoreType.DMA],
    )(_sc_body)
    return f(zrows, msg_flat, src_p, dst2d, ee)


def kernel(x, edge_index, edge_attr, params):
    p = params
    ei = edge_index.astype(jnp.int32)
    pad = NEP - NE
    src_p = jnp.concatenate([ei[0], jnp.zeros((pad,), jnp.int32)])
    dst2d = jnp.concatenate([ei[1], jnp.full((pad,), JUNK, jnp.int32)])
    dst2d = dst2d.reshape(NEP // 128, 128)
    zrows = jnp.zeros((TROWS, 16), jnp.float32)

    h = _lin(x, p['node_emb_W'], p['node_emb_b'][None, :])
    for l in range(3):
        wf = p['edge_emb_W'] @ p[f'l{l}_edge_W1']
        bf = p['edge_emb_b'] @ p[f'l{l}_edge_W1'] + p[f'l{l}_edge_b1']
        ee = _mlp_edges(edge_attr, wf, bf[None, :],
                        p[f'l{l}_edge_W2'], p[f'l{l}_edge_b2'][None, :])
        msg = _mlp_nodes(h, p[f'l{l}_node_W1'], p[f'l{l}_node_b1'][None, :],
                         p[f'l{l}_node_W2'], p[f'l{l}_node_b2'][None, :])
        parts = _sc_gms(zrows, msg.reshape(4 * N_NODES, 16), src_p, dst2d, ee)
        h = _comb(h, parts, p[f'l{l}_comb_W1'], p[f'l{l}_comb_b1'][None, :],
                  p[f'l{l}_comb_W2'], p[f'l{l}_comb_b2'][None, :])
    cs = _colsum(h)
    return _head(cs, p['ro_W1'], p['ro_b1'][None, :],
                 p['ro_W2'], p['ro_b2'][None, :],
                 p['out_W1'], p['out_b1'][None, :],
                 p['out_W2'], p['out_b2'][None, :])
